# baseline (speedup 1.0000x reference)
import jax
import jax.numpy as jnp
from jax import lax
from jax.experimental import pallas as pl
from jax.experimental.pallas import tpu as pltpu
from jax.experimental.pallas import tpu_sc as plsc

N = 10000
D = 128
K = 32
BN = 8
F = BN * K
NW = 32
NP = 10240
NR = NP // NW
PER_W = NP * K // NW
GCH = 320
NCH = PER_W // GCH
assert NCH * GCH == PER_W
CE = 4096
EP = 80 * CE
NV = CE // 16
PAD_ROW = 10200
PACK = 16384


def _matmul_body(x_ref, w_ref, o_ref):
    o_ref[...] = jnp.dot(x_ref[...], w_ref[...],
                         preferred_element_type=jnp.float32)


def _project(feat, W):
    return pl.pallas_call(
        _matmul_body,
        out_shape=jax.ShapeDtypeStruct((N, D), jnp.float32),
    )(feat, W)


def _topk_body(pk_hbm, idx_hbm, vf_hbm, pb0, pb1, slots, vfv, counts, s0, s1):
    i32 = jnp.int32
    wid = lax.axis_index("s") * 2 + lax.axis_index("c")
    lo = (wid * i32(NR)).astype(i32)
    iot = lax.iota(i32, 16)
    kk = i32(K)

    def _zero(ref, n16):
        def zbody(i, c):
            ref[pl.ds(i * 16, 16)] = jnp.zeros((16,), i32)
            return c
        lax.fori_loop(i32(0), i32(n16), zbody, i32(0), unroll=False)

    _zero(counts, NR // 16)
    _zero(slots, NR * K // 16)

    def process(buf):
        def vloop(v, carry):
            p = buf[pl.ds(v * 16, 16)]
            r = lax.shift_right_logical(p, i32(14))
            cl = p & i32(PACK - 1)
            local = r - lo
            m = (local >= 0) & (local < i32(NR))
            lclip = jnp.minimum(jnp.maximum(local, i32(0)), i32(NR - 1))
            cnt = plsc.load_gather(counts, [lclip], mask=m)
            dup, lastm = plsc.scan_count(r, mask=m)
            slot = cnt + dup - 1
            keep = m & (slot < kk)
            sclip = jnp.minimum(jnp.maximum(slot, i32(0)), kk - 1)
            addr = lclip * kk + sclip
            plsc.store_scatter(slots, [addr], cl, mask=keep)
            plsc.store_scatter(counts, [lclip], cnt + dup, mask=m & lastm)
            return carry
        lax.fori_loop(i32(0), i32(NV), vloop, i32(0), unroll=False)

    pltpu.async_copy(pk_hbm.at[pl.ds(i32(0), CE)], pb0, s0)
    pltpu.async_copy(pk_hbm.at[pl.ds(i32(CE), CE)], pb1, s1)
    npairs = EP // CE // 2

    def pair(c, carry):
        pltpu.make_async_copy(pk_hbm.at[pl.ds(i32(0), CE)], pb0, s0).wait()
        process(pb0)

        @pl.when(c < i32(npairs - 1))
        def _():
            pltpu.async_copy(
                pk_hbm.at[pl.ds((2 * c + 2) * i32(CE), CE)], pb0, s0)
        pltpu.make_async_copy(pk_hbm.at[pl.ds(i32(0), CE)], pb1, s1).wait()
        process(pb1)

        @pl.when(c < i32(npairs - 1))
        def _():
            pltpu.async_copy(
                pk_hbm.at[pl.ds((2 * c + 3) * i32(CE), CE)], pb1, s1)
        return carry

    lax.fori_loop(i32(0), i32(npairs), pair, i32(0), unroll=False)

    def selfloop(nb, carry):
        offs = nb * 16
        idsl = offs + iot
        cv = counts[pl.ds(offs, 16)]
        need = cv < kk
        addr = idsl * kk + jnp.minimum(cv, kk - 1)
        gid = jnp.minimum(lo + idsl, i32(N - 1))
        plsc.store_scatter(slots, [addr], gid, mask=need)
        counts[pl.ds(offs, 16)] = cv + need.astype(i32)
        return carry

    lax.fori_loop(i32(0), i32(NR // 16), selfloop, i32(0), unroll=False)

    def vfloop(i, carry):
        base = i * 16
        pos = base + iot
        nodev = lax.shift_right_logical(pos, i32(5))
        cv = plsc.load_gather(counts, [nodev])
        kpos = pos & i32(K - 1)
        vfv[pl.ds(base, 16)] = (kpos < cv).astype(jnp.float32)
        return carry

    lax.fori_loop(i32(0), i32(NR * K // 16), vfloop, i32(0), unroll=False)

    base_out = wid * i32(NR * K)
    pltpu.sync_copy(slots, idx_hbm.at[pl.ds(base_out, NR * K)])
    pltpu.sync_copy(vfv, vf_hbm.at[pl.ds(base_out, NR * K)])


def _topk_sc(packed):
    mesh = plsc.VectorSubcoreMesh(core_axis_name="c", subcore_axis_name="s")
    kfn = pl.kernel(
        _topk_body,
        out_type=(jax.ShapeDtypeStruct((NP * K,), jnp.int32),
                  jax.ShapeDtypeStruct((NP * K,), jnp.float32)),
        mesh=mesh,
        scratch_types=[
            pltpu.VMEM((CE,), jnp.int32),
            pltpu.VMEM((CE,), jnp.int32),
            pltpu.VMEM((NR * K,), jnp.int32),
            pltpu.VMEM((NR * K,), jnp.float32),
            pltpu.VMEM((NR,), jnp.int32),
            pltpu.SemaphoreType.DMA,
            pltpu.SemaphoreType.DMA,
        ],
        compiler_params=pltpu.CompilerParams(needs_layout_passes=False),
    )
    return kfn(packed)


def _gather_body(h_hbm, idx_hbm, g_hbm, idx_v, rows_v, sem):
    i32 = jnp.int32
    wid = lax.axis_index("s") * 2 + lax.axis_index("c")
    base = wid * i32(PER_W)

    def chunk(c, carry):
        off = base + c * i32(GCH)
        pltpu.sync_copy(idx_hbm.at[pl.ds(off, GCH)], idx_v)
        pltpu.async_copy(h_hbm.at[idx_v], rows_v, sem).wait()
        pltpu.sync_copy(rows_v, g_hbm.at[pl.ds(off, GCH)])
        return carry

    lax.fori_loop(i32(0), i32(NCH), chunk, i32(0), unroll=False)


def _gather(h, idx_flat):
    mesh = plsc.VectorSubcoreMesh(core_axis_name="c", subcore_axis_name="s")
    kfn = pl.kernel(
        _gather_body,
        out_type=jax.ShapeDtypeStruct((NP * K, D), jnp.float32),
        mesh=mesh,
        scratch_types=[
            pltpu.VMEM((GCH,), jnp.int32),
            pltpu.VMEM((GCH, D), jnp.float32),
            pltpu.SemaphoreType.DMA,
        ],
    )
    return kfn(h, idx_flat)


def _z():
    return jnp.int32(0)


def _softkconv_body(g_ref, vc_ref, vr_ref, b_ref, o_ref):
    f32 = jnp.float32
    G = g_ref[...]
    vc = vc_ref[...]
    vr = vr_ref[...].reshape(1, F)
    Q = G * G
    sqc = jnp.sum(Q, axis=1, keepdims=True)
    ones_r = jnp.ones((1, D), f32)
    tdim = (((1,), (1,)), ((), ()))
    sqr = lax.dot_general(ones_r, Q, tdim,
                          preferred_element_type=f32)
    inner = lax.dot_general(G, G, tdim,
                            preferred_element_type=f32)
    d2 = sqc + sqr - 2.0 * inner
    ia = lax.broadcasted_iota(jnp.int32, (F, F), 0)
    ib = lax.broadcasted_iota(jnp.int32, (F, F), 1)
    samen = (ia // K) == (ib // K)
    samenf = samen.astype(f32)
    offdiag = samenf * (1.0 - (ia == ib).astype(f32))
    dist = jnp.sqrt(jnp.maximum(d2, 0.0)) * (offdiag * vc * vr)
    distk = jnp.sum(dist, axis=1, keepdims=True)
    Bv = samenf * vr
    cntc = jnp.sum(Bv, axis=1, keepdims=True)
    ndim = (((1,), (0,)), ((), ()))
    Msum = lax.dot_general(Bv, distk, ndim,
                           preferred_element_type=f32)
    M = Msum / cntc
    z = jnp.where(vc > 0, M - distk, 0.0)
    e = jnp.where(vc > 0, jnp.exp(z), 0.0)
    S = lax.dot_general(samenf, e, ndim,
                        preferred_element_type=f32)
    rw = e / S
    ja = lax.broadcasted_iota(jnp.int32, (BN, F), 0)
    jb = lax.broadcasted_iota(jnp.int32, (BN, F), 1)
    Pm = (ja == (jb // K)).astype(f32)
    out = lax.dot_general(Pm, rw * G, ndim,
                          preferred_element_type=f32)
    o_ref[...] = out + b_ref[...]


def _softkconv(g, validc, validr, b):
    grid = N // BN
    return pl.pallas_call(
        _softkconv_body,
        grid=(grid,),
        in_specs=[
            pl.BlockSpec((F, D), lambda i: (i, _z())),
            pl.BlockSpec((F, 1), lambda i: (i, _z())),
            pl.BlockSpec((1, 1, F), lambda i: (i, _z(), _z())),
            pl.BlockSpec((1, D), lambda i: (_z(), _z())),
        ],
        out_specs=pl.BlockSpec((BN, D), lambda i: (i, _z())),
        out_shape=jax.ShapeDtypeStruct((N, D), jnp.float32),
    )(g, validc, validr, b)


def kernel(feat, edge_index, W, b):
    h = _project(feat, W)
    row = edge_index[0].astype(jnp.int32)
    col = edge_index[1].astype(jnp.int32)
    packed = row * PACK + col
    pad = jnp.full((EP - packed.shape[0],), PAD_ROW * PACK, jnp.int32)
    packed = jnp.concatenate([packed, pad])
    idx_pad, vf_pad = _topk_sc(packed)
    g_pad = _gather(h, idx_pad)
    validc = vf_pad.reshape(NP * K, 1)
    validr = vf_pad.reshape(NP // BN, 1, F)
    out = _softkconv(g_pad, validc, validr, b.reshape(1, D))
    return out


# speedup vs baseline: 1.1936x; 1.1936x over previous
import jax
import jax.numpy as jnp
from jax import lax
from jax.experimental import pallas as pl
from jax.experimental.pallas import tpu as pltpu
from jax.experimental.pallas import tpu_sc as plsc

N = 10000
D = 128
K = 32
BN = 8
F = BN * K
NW = 32
NP = 10240
NR = NP // NW
PER_W = NP * K // NW
GCH = 320
NCH = PER_W // GCH
assert NCH * GCH == PER_W
CE = 4096
EP = 80 * CE
NV = CE // 16
PAD_ROW = 10200
PACK = 16384


def _matmul_body(x_ref, w_ref, o_ref):
    o_ref[...] = jnp.dot(x_ref[...], w_ref[...],
                         preferred_element_type=jnp.float32)


def _project(feat, W):
    return pl.pallas_call(
        _matmul_body,
        out_shape=jax.ShapeDtypeStruct((N, D), jnp.float32),
    )(feat, W)


def _topk_body(pk_hbm, idx_hbm, vf_hbm, pb0, pb1, slots, vfv, counts, s0, s1):
    i32 = jnp.int32
    wid = lax.axis_index("s") * 2 + lax.axis_index("c")
    lo = (wid * i32(NR)).astype(i32)
    iot = lax.iota(i32, 16)
    kk = i32(K)

    def _zero(ref, n16):
        def zbody(i, c):
            ref[pl.ds(i * 16, 16)] = jnp.zeros((16,), i32)
            return c
        lax.fori_loop(i32(0), i32(n16), zbody, i32(0), unroll=False)

    _zero(counts, NR // 16)
    _zero(slots, NR * K // 16)

    def process(buf):
        def vloop(v, carry):
            p = buf[pl.ds(v * 16, 16)]
            r = lax.shift_right_logical(p, i32(14))
            cl = p & i32(PACK - 1)
            local = r - lo
            m = (local >= 0) & (local < i32(NR))
            lclip = jnp.minimum(jnp.maximum(local, i32(0)), i32(NR - 1))
            cnt = plsc.load_gather(counts, [lclip], mask=m)
            dup, lastm = plsc.scan_count(r, mask=m)
            slot = cnt + dup - 1
            keep = m & (slot < kk)
            sclip = jnp.minimum(jnp.maximum(slot, i32(0)), kk - 1)
            addr = lclip * kk + sclip
            plsc.store_scatter(slots, [addr], cl, mask=keep)
            plsc.store_scatter(counts, [lclip], cnt + dup, mask=m & lastm)
            return carry
        lax.fori_loop(i32(0), i32(NV), vloop, i32(0), unroll=False)

    pltpu.async_copy(pk_hbm.at[pl.ds(i32(0), CE)], pb0, s0)
    pltpu.async_copy(pk_hbm.at[pl.ds(i32(CE), CE)], pb1, s1)
    npairs = EP // CE // 2

    def pair(c, carry):
        pltpu.make_async_copy(pk_hbm.at[pl.ds(i32(0), CE)], pb0, s0).wait()
        process(pb0)

        @pl.when(c < i32(npairs - 1))
        def _():
            pltpu.async_copy(
                pk_hbm.at[pl.ds((2 * c + 2) * i32(CE), CE)], pb0, s0)
        pltpu.make_async_copy(pk_hbm.at[pl.ds(i32(0), CE)], pb1, s1).wait()
        process(pb1)

        @pl.when(c < i32(npairs - 1))
        def _():
            pltpu.async_copy(
                pk_hbm.at[pl.ds((2 * c + 3) * i32(CE), CE)], pb1, s1)
        return carry

    lax.fori_loop(i32(0), i32(npairs), pair, i32(0), unroll=False)

    def selfloop(nb, carry):
        offs = nb * 16
        idsl = offs + iot
        cv = counts[pl.ds(offs, 16)]
        need = cv < kk
        addr = idsl * kk + jnp.minimum(cv, kk - 1)
        gid = jnp.minimum(lo + idsl, i32(N - 1))
        plsc.store_scatter(slots, [addr], gid, mask=need)
        counts[pl.ds(offs, 16)] = cv + need.astype(i32)
        return carry

    lax.fori_loop(i32(0), i32(NR // 16), selfloop, i32(0), unroll=False)

    def vfloop(i, carry):
        base = i * 16
        pos = base + iot
        nodev = lax.shift_right_logical(pos, i32(5))
        cv = plsc.load_gather(counts, [nodev])
        kpos = pos & i32(K - 1)
        vfv[pl.ds(base, 16)] = (kpos < cv).astype(jnp.float32)
        return carry

    lax.fori_loop(i32(0), i32(NR * K // 16), vfloop, i32(0), unroll=False)

    base_out = wid * i32(NR * K)
    pltpu.sync_copy(slots, idx_hbm.at[pl.ds(base_out, NR * K)])
    pltpu.sync_copy(vfv, vf_hbm.at[pl.ds(base_out, NR * K)])


def _topk_sc(packed):
    mesh = plsc.VectorSubcoreMesh(core_axis_name="c", subcore_axis_name="s")
    kfn = pl.kernel(
        _topk_body,
        out_type=(jax.ShapeDtypeStruct((NP * K,), jnp.int32),
                  jax.ShapeDtypeStruct((NP * K,), jnp.float32)),
        mesh=mesh,
        scratch_types=[
            pltpu.VMEM((CE,), jnp.int32),
            pltpu.VMEM((CE,), jnp.int32),
            pltpu.VMEM((NR * K,), jnp.int32),
            pltpu.VMEM((NR * K,), jnp.float32),
            pltpu.VMEM((NR,), jnp.int32),
            pltpu.SemaphoreType.DMA,
            pltpu.SemaphoreType.DMA,
        ],
        compiler_params=pltpu.CompilerParams(needs_layout_passes=False),
    )
    return kfn(packed)


def _gather_body(h_hbm, idx_hbm, g_hbm, idx_v, rows_v, sem):
    i32 = jnp.int32
    wid = lax.axis_index("s") * 2 + lax.axis_index("c")
    base = wid * i32(PER_W)

    def chunk(c, carry):
        off = base + c * i32(GCH)

        @pl.when(off < i32(N * K))
        def _():
            pltpu.sync_copy(idx_hbm.at[pl.ds(off, GCH)], idx_v)
            pltpu.async_copy(h_hbm.at[idx_v], rows_v, sem).wait()
            pltpu.sync_copy(rows_v, g_hbm.at[pl.ds(off, GCH)])
        return carry

    lax.fori_loop(i32(0), i32(NCH), chunk, i32(0), unroll=False)


def _gather(h, idx_flat):
    mesh = plsc.VectorSubcoreMesh(core_axis_name="c", subcore_axis_name="s")
    kfn = pl.kernel(
        _gather_body,
        out_type=jax.ShapeDtypeStruct((NP * K, D), jnp.float32),
        mesh=mesh,
        scratch_types=[
            pltpu.VMEM((GCH,), jnp.int32),
            pltpu.VMEM((GCH, D), jnp.float32),
            pltpu.SemaphoreType.DMA,
        ],
    )
    return kfn(h, idx_flat)


def _z():
    return jnp.int32(0)


def _softkconv_half(G, vc, vr, b, o_ref, half):
    f32 = jnp.float32
    Q = G * G
    sqc = jnp.sum(Q, axis=1, keepdims=True)
    ones_r = jnp.ones((1, D), f32)
    tdim = (((1,), (1,)), ((), ()))
    sqr = lax.dot_general(ones_r, Q, tdim,
                          preferred_element_type=f32)
    inner = lax.dot_general(G, G, tdim,
                            preferred_element_type=f32)
    d2 = sqc + sqr - 2.0 * inner
    ia = lax.broadcasted_iota(jnp.int32, (F, F), 0)
    ib = lax.broadcasted_iota(jnp.int32, (F, F), 1)
    samen = (ia // K) == (ib // K)
    samenf = samen.astype(f32)
    offdiag = samenf * (1.0 - (ia == ib).astype(f32))
    dist = jnp.sqrt(jnp.maximum(d2, 0.0)) * (offdiag * vc * vr)
    distk = jnp.sum(dist, axis=1, keepdims=True)
    Bv = samenf * vr
    cntc = jnp.sum(Bv, axis=1, keepdims=True)
    ndim = (((1,), (0,)), ((), ()))
    Msum = lax.dot_general(Bv, distk, ndim,
                           preferred_element_type=f32)
    M = Msum / cntc
    z = jnp.where(vc > 0, M - distk, 0.0)
    e = jnp.where(vc > 0, jnp.exp(z), 0.0)
    S = lax.dot_general(samenf, e, ndim,
                        preferred_element_type=f32)
    rw = e / S
    ja = lax.broadcasted_iota(jnp.int32, (BN, F), 0)
    jb = lax.broadcasted_iota(jnp.int32, (BN, F), 1)
    Pm = (ja == (jb // K)).astype(f32)
    out = lax.dot_general(Pm, rw * G, ndim,
                          preferred_element_type=f32)
    o_ref[half * BN:(half + 1) * BN, :] = out + b


def _softkconv_body(g_ref, vc_ref, vr_ref, b_ref, o_ref):
    b = b_ref[...]
    vr = vr_ref[...].reshape(1, 2 * F)
    for half in range(2):
        s = half * F
        _softkconv_half(g_ref[s:s + F, :], vc_ref[s:s + F, :],
                        vr[:, s:s + F], b, o_ref, half)


def _softkconv(g, validc, validr, b):
    grid = N // (2 * BN)
    return pl.pallas_call(
        _softkconv_body,
        grid=(grid,),
        in_specs=[
            pl.BlockSpec((2 * F, D), lambda i: (i, _z())),
            pl.BlockSpec((2 * F, 1), lambda i: (i, _z())),
            pl.BlockSpec((1, 1, 2 * F), lambda i: (i, _z(), _z())),
            pl.BlockSpec((1, D), lambda i: (_z(), _z())),
        ],
        out_specs=pl.BlockSpec((2 * BN, D), lambda i: (i, _z())),
        out_shape=jax.ShapeDtypeStruct((N, D), jnp.float32),
    )(g, validc, validr, b)


def kernel(feat, edge_index, W, b):
    h = _project(feat, W)
    row = edge_index[0].astype(jnp.int32)
    col = edge_index[1].astype(jnp.int32)
    packed = row * PACK + col
    pad = jnp.full((EP - packed.shape[0],), PAD_ROW * PACK, jnp.int32)
    packed = jnp.concatenate([packed, pad])
    idx_pad, vf_pad = _topk_sc(packed)
    g_pad = _gather(h, idx_pad)
    validc = vf_pad.reshape(NP * K, 1)
    validr = vf_pad.reshape(NP // (2 * BN), 1, 2 * F)
    out = _softkconv(g_pad, validc, validr, b.reshape(1, D))
    return out
